# Initial kernel scaffold; baseline (speedup 1.0000x reference)
import jax
import jax.numpy as jnp
from jax.experimental import pallas as pl


def kernel(x, length, pe, se):
    raise NotImplementedError("write your pallas kernel here")


# speedup vs baseline: 4.3209x; 4.3209x over previous
import jax
import jax.numpy as jnp
from jax.experimental import pallas as pl
from jax.experimental.pallas import tpu as pltpu

_SBLK = 256


def _pe_add_body(end_ref, x_ref, pe_ref, se_ref, o_ref):
    si = pl.program_id(0)
    b = pl.program_id(1)
    s0 = si * _SBLK
    rows = jax.lax.broadcasted_iota(jnp.int32, (_SBLK, 1), 0) + s0
    end_b = end_ref[b]
    se_sel = jnp.where(rows < end_b, se_ref[0, :][None, :], se_ref[1, :][None, :])
    se_sel = jnp.where(rows == 0, jnp.zeros_like(se_sel), se_sel)
    o_ref[0] = x_ref[0] + pe_ref[0] + se_sel


def kernel(x, length, pe, se):
    B, S, D = x.shape
    pe_pad = jnp.concatenate([jnp.zeros((1, 1, D), x.dtype), pe], axis=1)
    end = (1 + length).astype(jnp.int32)
    grid_spec = pltpu.PrefetchScalarGridSpec(
        num_scalar_prefetch=1,
        grid=(S // _SBLK, B),
        in_specs=[
            pl.BlockSpec((1, _SBLK, D), lambda si, b, end_ref: (b, si, 0)),
            pl.BlockSpec((1, _SBLK, D), lambda si, b, end_ref: (0, si, 0)),
            pl.BlockSpec((2, D), lambda si, b, end_ref: (0, 0)),
        ],
        out_specs=pl.BlockSpec((1, _SBLK, D), lambda si, b, end_ref: (b, si, 0)),
    )
    return pl.pallas_call(
        _pe_add_body,
        grid_spec=grid_spec,
        out_shape=jax.ShapeDtypeStruct((B, S, D), x.dtype),
    )(end, x, pe_pad, se)


# speedup vs baseline: 5.5814x; 1.2917x over previous
import jax
import jax.numpy as jnp
from jax.experimental import pallas as pl
from jax.experimental.pallas import tpu as pltpu

_SBLK = 512


def _pe_add_body(end_ref, x_ref, pe_ref, se_ref, o_ref):
    si = pl.program_id(0)
    b = pl.program_id(1)
    s0 = si * _SBLK
    rows = jax.lax.broadcasted_iota(jnp.int32, (_SBLK, 1), 0) + s0
    end_b = end_ref[b]
    se_sel = jnp.where(rows < end_b, se_ref[0, :][None, :], se_ref[1, :][None, :])
    se_sel = jnp.where(rows == 0, jnp.zeros_like(se_sel), se_sel)
    o_ref[0] = x_ref[0] + pe_ref[0] + se_sel


def kernel(x, length, pe, se):
    B, S, D = x.shape
    pe_pad = jnp.concatenate([jnp.zeros((1, 1, D), x.dtype), pe], axis=1)
    end = (1 + length).astype(jnp.int32)
    grid_spec = pltpu.PrefetchScalarGridSpec(
        num_scalar_prefetch=1,
        grid=(S // _SBLK, B),
        in_specs=[
            pl.BlockSpec((1, _SBLK, D), lambda si, b, end_ref: (b, si, 0)),
            pl.BlockSpec((1, _SBLK, D), lambda si, b, end_ref: (0, si, 0)),
            pl.BlockSpec((2, D), lambda si, b, end_ref: (0, 0)),
        ],
        out_specs=pl.BlockSpec((1, _SBLK, D), lambda si, b, end_ref: (b, si, 0)),
    )
    return pl.pallas_call(
        _pe_add_body,
        grid_spec=grid_spec,
        out_shape=jax.ShapeDtypeStruct((B, S, D), x.dtype),
    )(end, x, pe_pad, se)


# speedup vs baseline: 6.0297x; 1.0803x over previous
import jax
import jax.numpy as jnp
from jax.experimental import pallas as pl
from jax.experimental.pallas import tpu as pltpu

_SBLK = 1024


def _pe_add_body(end_ref, x_ref, pe_ref, se_ref, o_ref):
    si = pl.program_id(0)
    b = pl.program_id(1)
    s0 = si * _SBLK
    rows = jax.lax.broadcasted_iota(jnp.int32, (_SBLK, 1), 0) + s0
    end_b = end_ref[b]
    se_sel = jnp.where(rows < end_b, se_ref[0, :][None, :], se_ref[1, :][None, :])
    se_sel = jnp.where(rows == 0, jnp.zeros_like(se_sel), se_sel)
    o_ref[0] = x_ref[0] + pe_ref[0] + se_sel


def kernel(x, length, pe, se):
    B, S, D = x.shape
    pe_pad = jnp.concatenate([jnp.zeros((1, 1, D), x.dtype), pe], axis=1)
    end = (1 + length).astype(jnp.int32)
    grid_spec = pltpu.PrefetchScalarGridSpec(
        num_scalar_prefetch=1,
        grid=(S // _SBLK, B),
        in_specs=[
            pl.BlockSpec((1, _SBLK, D), lambda si, b, end_ref: (b, si, 0)),
            pl.BlockSpec((1, _SBLK, D), lambda si, b, end_ref: (0, si, 0)),
            pl.BlockSpec((2, D), lambda si, b, end_ref: (0, 0)),
        ],
        out_specs=pl.BlockSpec((1, _SBLK, D), lambda si, b, end_ref: (b, si, 0)),
    )
    return pl.pallas_call(
        _pe_add_body,
        grid_spec=grid_spec,
        out_shape=jax.ShapeDtypeStruct((B, S, D), x.dtype),
    )(end, x, pe_pad, se)


# speedup vs baseline: 6.2077x; 1.0295x over previous
import jax
import jax.numpy as jnp
from jax.experimental import pallas as pl
from jax.experimental.pallas import tpu as pltpu

_SBLK = 2048


def _pe_add_body(end_ref, x_ref, pe_ref, se_ref, o_ref):
    si = pl.program_id(0)
    b = pl.program_id(1)
    s0 = si * _SBLK
    rows = jax.lax.broadcasted_iota(jnp.int32, (_SBLK, 1), 0) + s0
    end_b = end_ref[b]
    se_sel = jnp.where(rows < end_b, se_ref[0, :][None, :], se_ref[1, :][None, :])
    se_sel = jnp.where(rows == 0, jnp.zeros_like(se_sel), se_sel)
    o_ref[0] = x_ref[0] + pe_ref[0] + se_sel


def kernel(x, length, pe, se):
    B, S, D = x.shape
    pe_pad = jnp.concatenate([jnp.zeros((1, 1, D), x.dtype), pe], axis=1)
    end = (1 + length).astype(jnp.int32)
    grid_spec = pltpu.PrefetchScalarGridSpec(
        num_scalar_prefetch=1,
        grid=(S // _SBLK, B),
        in_specs=[
            pl.BlockSpec((1, _SBLK, D), lambda si, b, end_ref: (b, si, 0)),
            pl.BlockSpec((1, _SBLK, D), lambda si, b, end_ref: (0, si, 0)),
            pl.BlockSpec((2, D), lambda si, b, end_ref: (0, 0)),
        ],
        out_specs=pl.BlockSpec((1, _SBLK, D), lambda si, b, end_ref: (b, si, 0)),
    )
    return pl.pallas_call(
        _pe_add_body,
        grid_spec=grid_spec,
        out_shape=jax.ShapeDtypeStruct((B, S, D), x.dtype),
    )(end, x, pe_pad, se)
